# Initial kernel scaffold; baseline (speedup 1.0000x reference)
import jax
import jax.numpy as jnp
from jax.experimental import pallas as pl


def kernel(H_input):
    raise NotImplementedError("write your pallas kernel here")


# speedup vs baseline: 18.8997x; 18.8997x over previous
import jax
import jax.numpy as jnp
from jax import lax
from jax.experimental import pallas as pl
from jax.experimental.pallas import tpu as pltpu

_N = 1024
_NR = 512
_NW = 16


def _gj_body(hin_ref, g_ref, src_ref, hp_ref, cs_ref, u_ref, m_ref):
    i32 = jnp.int32
    f32 = jnp.float32

    sh32 = lax.broadcasted_iota(i32, (32, 1), 0)
    for w in range(_NW):
        blk = hin_ref[pl.ds(32 * w, 32), :].astype(i32)
        hp_ref[pl.ds(w, 1), :] = jnp.sum(blk << sh32, axis=0, keepdims=True)

    cs_ref[0:1, :] = lax.broadcasted_iota(i32, (1, _N), 1)

    iota_w = lax.broadcasted_iota(i32, (_NW, 1), 0)
    bit_iota = lax.broadcasted_iota(i32, (_NW, 32), 1)
    rowid = lax.broadcasted_iota(i32, (_NW, 32), 0) * 32 + bit_iota
    lane_n = lax.broadcasted_iota(i32, (1, _N), 1)


    def p1_body(i, pivot):
        col = _N - 1 - i
        H = hp_ref[:, :]
        c16 = jnp.sum(jnp.where(lane_n == col, H, 0),
                      axis=1, keepdims=True)
        bits = (c16 >> bit_iota) & 1
        cand = jnp.where((bits != 0) & (rowid < pivot), rowid, -1)
        max_row = jnp.max(cand)
        found = (max_row >= 0) & (pivot > 0)
        new_pivot = pivot - 1
        r1 = jnp.where(found, max_row, 0)
        r2 = jnp.where(found, new_pivot, 0)
        w1, b1 = r1 >> 5, r1 & 31
        w2, b2 = r2 >> 5, r2 & 31
        row1 = jnp.sum(jnp.where(iota_w == w1, H, 0),
                       axis=0, keepdims=True)
        row2 = jnp.sum(jnp.where(iota_w == w2, H, 0),
                       axis=0, keepdims=True)
        d = ((row1 >> b1) ^ (row2 >> b2)) & 1
        d = jnp.where(found, d, 0)
        H = (H ^ jnp.where(iota_w == w1, d << b1, 0)
               ^ jnp.where(iota_w == w2, d << b2, 0))
        rowmask = 0 - ((row1 >> b1) & 1)
        clr1 = jnp.where(iota_w == w1, i32(1) << b1, 0)
        clr2 = jnp.where(iota_w == w2, i32(1) << b2, 0)
        pc16 = jnp.where(found, c16 & ~clr1 & ~clr2, 0)
        hp_ref[:, :] = H ^ (pc16 & rowmask)
        return jnp.where(found, new_pivot, pivot)

    lax.fori_loop(0, _N, p1_body, i32(_NR))

    def p2_body(i, carry):
        ew, eb = i >> 5, i & 31
        t16 = jnp.where(iota_w == ew, i32(1) << eb, 0)
        H = hp_ref[:, :]
        diff_cnt = jnp.sum(jnp.where((H ^ t16) != 0, 1, 0),
                           axis=0, keepdims=True)
        matchv = diff_cnt == 0
        cond = jnp.any(matchv)
        jstar = jnp.min(jnp.where(matchv, lane_n, i32(1) << 20))
        jstar = jnp.where(cond, jstar, 0)
        cei = _N - _NR + i
        colA = jnp.sum(jnp.where(lane_n == jstar, H, 0),
                       axis=1, keepdims=True)
        colB = jnp.sum(jnp.where(lane_n == cei, H, 0),
                       axis=1, keepdims=True)
        dcol = jnp.where(cond, colA ^ colB, 0)
        swap_lanes = (lane_n == jstar) | (lane_n == cei)
        hp_ref[:, :] = H ^ jnp.where(swap_lanes, dcol, 0)
        csv = cs_ref[0:1, :]
        sA = jnp.sum(jnp.where(lane_n == jstar, csv, 0))
        sB = jnp.sum(jnp.where(lane_n == cei, csv, 0))
        csn = jnp.where(lane_n == jstar, sB,
                        jnp.where(lane_n == cei, sA, csv))
        cs_ref[0:1, :] = jnp.where(cond, csn, csv)
        return carry

    lax.fori_loop(0, _NR, p2_body, i32(0))

    for w in range(_NW):
        rw = hp_ref[pl.ds(w, 1), 0:_NR]
        u_ref[pl.ds(32 * w, 32), :] = ((rw >> sh32) & 1).astype(f32)

    csf = cs_ref[0:1, :].astype(f32)
    eye = (lax.broadcasted_iota(i32, (_NR, _NR), 0)
           == lax.broadcasted_iota(i32, (_NR, _NR), 1)).astype(f32)
    cs_top = lax.dot_general(eye, csf[:, 0:_NR],
                             (((1,), (1,)), ((), ())),
                             preferred_element_type=f32)
    cs_bot = lax.dot_general(eye, csf[:, _NR:_N],
                             (((1,), (1,)), ((), ())),
                             preferred_element_type=f32)
    lane_ni = lax.broadcasted_iota(i32, (_NR, _N), 1)
    m_ref[:, :] = (cs_bot.astype(i32) == lane_ni).astype(f32)
    g_ref[:, :] = ((cs_top.astype(i32) == lane_ni).astype(f32)
                   + lax.dot_general(u_ref[:, :], m_ref[:, :],
                                     (((0,), (0,)), ((), ())),
                                     preferred_element_type=f32))
    src_ref[0:1, :] = cs_ref[0:1, 0:_NR]


def kernel(H_input):
    g, src = pl.pallas_call(
        _gj_body,
        out_shape=[
            jax.ShapeDtypeStruct((_NR, _N), jnp.float32),
            jax.ShapeDtypeStruct((1, _NR), jnp.int32),
        ],
        scratch_shapes=[
            pltpu.VMEM((_NW, _N), jnp.int32),
            pltpu.VMEM((1, _N), jnp.int32),
            pltpu.VMEM((_NR, _NR), jnp.float32),
            pltpu.VMEM((_NR, _N), jnp.float32),
        ],
    )(H_input)
    return g, src.reshape(_NR)


# speedup vs baseline: 19.1008x; 1.0106x over previous
import jax
import jax.numpy as jnp
from jax import lax
from jax.experimental import pallas as pl
from jax.experimental.pallas import tpu as pltpu

_N = 1024
_NR = 512
_NW = 16


def _gj_body(hin_ref, g_ref, src_ref, hp_ref, cs_ref, u_ref, m_ref):
    i32 = jnp.int32
    f32 = jnp.float32

    sh32 = lax.broadcasted_iota(i32, (32, 1), 0)
    for w in range(_NW):
        blk = hin_ref[pl.ds(32 * w, 32), :].astype(i32)
        hp_ref[pl.ds(w, 1), :] = jnp.sum(blk << sh32, axis=0, keepdims=True)

    cs_ref[0:1, :] = lax.broadcasted_iota(i32, (1, _N), 1)

    iota_w = lax.broadcasted_iota(i32, (_NW, 1), 0)
    bit_iota = lax.broadcasted_iota(i32, (_NW, 32), 1)
    rowid = lax.broadcasted_iota(i32, (_NW, 32), 0) * 32 + bit_iota
    lane_n = lax.broadcasted_iota(i32, (1, _N), 1)


    def p1_body(i, carry):
        H, pivot = carry
        col = _N - 1 - i
        c16 = jnp.sum(jnp.where(lane_n == col, H, 0),
                      axis=1, keepdims=True)
        bits = (c16 >> bit_iota) & 1
        cand = jnp.where((bits != 0) & (rowid < pivot), rowid, -1)
        max_row = jnp.max(cand, axis=(0, 1), keepdims=True)[:1, :1]
        found = (max_row >= 0) & (pivot > 0)
        new_pivot = pivot - 1
        r1 = jnp.where(found, max_row, 0)
        r2 = jnp.where(found, new_pivot, 0)
        w1, b1 = r1 >> 5, r1 & 31
        w2, b2 = r2 >> 5, r2 & 31
        comb = jnp.sum(jnp.where(iota_w == w1, (H >> b1) & 1, 0)
                       + jnp.where(iota_w == w2, ((H >> b2) & 1) << 1, 0),
                       axis=0, keepdims=True)
        r1bit = comb & 1
        d = jnp.where(found, (comb ^ (comb >> 1)) & 1, 0)
        H = (H ^ jnp.where(iota_w == w1, d << b1, 0)
               ^ jnp.where(iota_w == w2, d << b2, 0))
        rowmask = 0 - r1bit
        clr1 = jnp.where(iota_w == w1, i32(1) << b1, 0)
        clr2 = jnp.where(iota_w == w2, i32(1) << b2, 0)
        pc16 = jnp.where(found, c16 & ~clr1 & ~clr2, 0)
        H = H ^ (pc16 & rowmask)
        return H, jnp.where(found, new_pivot, pivot)

    H0 = hp_ref[:, :]
    piv0 = jnp.full((1, 1), _NR, dtype=i32)
    Hf, _ = lax.fori_loop(0, _N, p1_body, (H0, piv0))

    def p2_body(i, carry):
        H, csv = carry
        ew, eb = i >> 5, i & 31
        t16 = jnp.where(iota_w == ew, i32(1) << eb, 0)
        diff_cnt = jnp.sum(jnp.where((H ^ t16) != 0, 1, 0),
                           axis=0, keepdims=True)
        matchv = diff_cnt == 0
        cond = jnp.any(matchv, axis=(0, 1), keepdims=True)[:1, :1]
        jstar = jnp.min(jnp.where(matchv, lane_n, i32(1) << 20),
                        axis=(0, 1), keepdims=True)[:1, :1]
        jstar = jnp.where(cond, jstar, 0)
        cei = _N - _NR + i
        mA = lane_n == jstar
        mB = lane_n == cei
        colA = jnp.sum(jnp.where(mA, H, 0), axis=1, keepdims=True)
        colB = jnp.sum(jnp.where(mB, H, 0), axis=1, keepdims=True)
        dcol = jnp.where(cond, colA ^ colB, 0)
        H = H ^ jnp.where(mA | mB, dcol, 0)
        sA = jnp.sum(jnp.where(mA, csv, 0), axis=(0, 1),
                     keepdims=True)[:1, :1]
        sB = jnp.sum(jnp.where(mB, csv, 0), axis=(0, 1),
                     keepdims=True)[:1, :1]
        csn = jnp.where(mA, sB, jnp.where(mB, sA, csv))
        return H, jnp.where(cond, csn, csv)

    Hf, csf_v = lax.fori_loop(0, _NR, p2_body,
                              (Hf, lax.broadcasted_iota(i32, (1, _N), 1)))
    hp_ref[:, :] = Hf
    cs_ref[0:1, :] = csf_v

    for w in range(_NW):
        rw = hp_ref[pl.ds(w, 1), 0:_NR]
        u_ref[pl.ds(32 * w, 32), :] = ((rw >> sh32) & 1).astype(f32)

    csf = cs_ref[0:1, :].astype(f32)
    eye = (lax.broadcasted_iota(i32, (_NR, _NR), 0)
           == lax.broadcasted_iota(i32, (_NR, _NR), 1)).astype(f32)
    cs_top = lax.dot_general(eye, csf[:, 0:_NR],
                             (((1,), (1,)), ((), ())),
                             preferred_element_type=f32)
    cs_bot = lax.dot_general(eye, csf[:, _NR:_N],
                             (((1,), (1,)), ((), ())),
                             preferred_element_type=f32)
    lane_ni = lax.broadcasted_iota(i32, (_NR, _N), 1)
    m_ref[:, :] = (cs_bot.astype(i32) == lane_ni).astype(f32)
    g_ref[:, :] = ((cs_top.astype(i32) == lane_ni).astype(f32)
                   + lax.dot_general(u_ref[:, :], m_ref[:, :],
                                     (((0,), (0,)), ((), ())),
                                     preferred_element_type=f32))
    src_ref[0:1, :] = cs_ref[0:1, 0:_NR]


def kernel(H_input):
    g, src = pl.pallas_call(
        _gj_body,
        out_shape=[
            jax.ShapeDtypeStruct((_NR, _N), jnp.float32),
            jax.ShapeDtypeStruct((1, _NR), jnp.int32),
        ],
        scratch_shapes=[
            pltpu.VMEM((_NW, _N), jnp.int32),
            pltpu.VMEM((1, _N), jnp.int32),
            pltpu.VMEM((_NR, _NR), jnp.float32),
            pltpu.VMEM((_NR, _N), jnp.float32),
        ],
    )(H_input)
    return g, src.reshape(_NR)


# speedup vs baseline: 28.5329x; 1.4938x over previous
import jax
import jax.numpy as jnp
from jax import lax
from jax.experimental import pallas as pl
from jax.experimental.pallas import tpu as pltpu

_N = 1024
_NR = 512
_NW = 16


def _gj_body(hin_ref, g_ref, src_ref, hp_ref, cs_ref, u_ref, m_ref):
    i32 = jnp.int32
    f32 = jnp.float32

    sh32 = lax.broadcasted_iota(i32, (32, 1), 0)
    for w in range(_NW):
        blk = hin_ref[pl.ds(32 * w, 32), :].astype(i32)
        hp_ref[pl.ds(w, 1), :] = jnp.sum(blk << sh32, axis=0, keepdims=True)

    cs_ref[0:1, :] = lax.broadcasted_iota(i32, (1, _N), 1)

    iota_w = lax.broadcasted_iota(i32, (_NW, 1), 0)
    bit_iota = lax.broadcasted_iota(i32, (_NW, 32), 1)
    rowid = lax.broadcasted_iota(i32, (_NW, 32), 0) * 32 + bit_iota
    lane_n = lax.broadcasted_iota(i32, (1, _N), 1)


    def p1_cond(carry):
        i, H, pivot = carry
        return (i < _N) & (pivot > 0)

    def p1_step(carry):
        i, H, pivot = carry
        col = _N - 1 - i
        c16 = jnp.sum(jnp.where(lane_n == col, H, 0),
                      axis=1, keepdims=True)
        bits = (c16 >> bit_iota) & 1
        cand = jnp.where((bits != 0) & (rowid < pivot), rowid, -1)
        max_row = jnp.max(cand)
        found = max_row >= 0
        new_pivot = pivot - 1
        r1 = jnp.where(found, max_row, 0)
        r2 = jnp.where(found, new_pivot, 0)
        w1, b1 = r1 >> 5, r1 & 31
        w2, b2 = r2 >> 5, r2 & 31
        comb = jnp.sum(jnp.where(iota_w == w1, (H >> b1) & 1, 0)
                       + jnp.where(iota_w == w2, ((H >> b2) & 1) << 1, 0),
                       axis=0, keepdims=True)
        r1bit = comb & 1
        d = jnp.where(found, (comb ^ (comb >> 1)) & 1, 0)
        H = (H ^ jnp.where(iota_w == w1, d << b1, 0)
               ^ jnp.where(iota_w == w2, d << b2, 0))
        rowmask = 0 - r1bit
        clr1 = jnp.where(iota_w == w1, i32(1) << b1, 0)
        clr2 = jnp.where(iota_w == w2, i32(1) << b2, 0)
        pc16 = jnp.where(found, c16 & ~clr1 & ~clr2, 0)
        H = H ^ (pc16 & rowmask)
        return i + 1, H, jnp.where(found, new_pivot, pivot)

    H0 = hp_ref[:, :]
    _, Hf, _ = lax.while_loop(p1_cond, p1_step, (i32(0), H0, i32(_NR)))

    def p2_body(i, carry):
        H, csv = carry
        ew, eb = i >> 5, i & 31
        t16 = jnp.where(iota_w == ew, i32(1) << eb, 0)
        diff_cnt = jnp.sum(jnp.where((H ^ t16) != 0, 1, 0),
                           axis=0, keepdims=True)
        matchv = diff_cnt == 0
        cond = jnp.any(matchv, axis=(0, 1), keepdims=True)[:1, :1]
        jstar = jnp.min(jnp.where(matchv, lane_n, i32(1) << 20),
                        axis=(0, 1), keepdims=True)[:1, :1]
        jstar = jnp.where(cond, jstar, 0)
        cei = _N - _NR + i
        mA = lane_n == jstar
        mB = lane_n == cei
        colA = jnp.sum(jnp.where(mA, H, 0), axis=1, keepdims=True)
        colB = jnp.sum(jnp.where(mB, H, 0), axis=1, keepdims=True)
        dcol = jnp.where(cond, colA ^ colB, 0)
        H = H ^ jnp.where(mA | mB, dcol, 0)
        sA = jnp.sum(jnp.where(mA, csv, 0), axis=(0, 1),
                     keepdims=True)[:1, :1]
        sB = jnp.sum(jnp.where(mB, csv, 0), axis=(0, 1),
                     keepdims=True)[:1, :1]
        csn = jnp.where(mA, sB, jnp.where(mB, sA, csv))
        return H, jnp.where(cond, csn, csv)

    Hf, csf_v = lax.fori_loop(0, _NR, p2_body,
                              (Hf, lax.broadcasted_iota(i32, (1, _N), 1)))
    hp_ref[:, :] = Hf
    cs_ref[0:1, :] = csf_v

    for w in range(_NW):
        rw = hp_ref[pl.ds(w, 1), 0:_NR]
        u_ref[pl.ds(32 * w, 32), :] = ((rw >> sh32) & 1).astype(f32)

    csf = cs_ref[0:1, :].astype(f32)
    eye = (lax.broadcasted_iota(i32, (_NR, _NR), 0)
           == lax.broadcasted_iota(i32, (_NR, _NR), 1)).astype(f32)
    cs_top = lax.dot_general(eye, csf[:, 0:_NR],
                             (((1,), (1,)), ((), ())),
                             preferred_element_type=f32)
    cs_bot = lax.dot_general(eye, csf[:, _NR:_N],
                             (((1,), (1,)), ((), ())),
                             preferred_element_type=f32)
    lane_ni = lax.broadcasted_iota(i32, (_NR, _N), 1)
    m_ref[:, :] = (cs_bot.astype(i32) == lane_ni).astype(f32)
    g_ref[:, :] = ((cs_top.astype(i32) == lane_ni).astype(f32)
                   + lax.dot_general(u_ref[:, :], m_ref[:, :],
                                     (((0,), (0,)), ((), ())),
                                     preferred_element_type=f32))
    src_ref[0:1, :] = cs_ref[0:1, 0:_NR]


def kernel(H_input):
    g, src = pl.pallas_call(
        _gj_body,
        out_shape=[
            jax.ShapeDtypeStruct((_NR, _N), jnp.float32),
            jax.ShapeDtypeStruct((1, _NR), jnp.int32),
        ],
        scratch_shapes=[
            pltpu.VMEM((_NW, _N), jnp.int32),
            pltpu.VMEM((1, _N), jnp.int32),
            pltpu.VMEM((_NR, _NR), jnp.float32),
            pltpu.VMEM((_NR, _N), jnp.float32),
        ],
    )(H_input)
    return g, src.reshape(_NR)


# speedup vs baseline: 29.0330x; 1.0175x over previous
import jax
import jax.numpy as jnp
from jax import lax
from jax.experimental import pallas as pl
from jax.experimental.pallas import tpu as pltpu

_N = 1024
_NR = 512
_NW = 16


def _gj_body(hin_ref, g_ref, src_ref, u_ref, m_ref):
    i32 = jnp.int32
    f32 = jnp.float32

    sh32 = lax.broadcasted_iota(i32, (32, 1), 0)
    H0 = jnp.concatenate(
        [jnp.sum(hin_ref[pl.ds(32 * w, 32), :].astype(i32) << sh32,
                 axis=0, keepdims=True) for w in range(_NW)], axis=0)

    iota_w = lax.broadcasted_iota(i32, (_NW, 1), 0)
    bit_iota = lax.broadcasted_iota(i32, (_NW, 32), 1)
    rowid = lax.broadcasted_iota(i32, (_NW, 32), 0) * 32 + bit_iota
    lane_n = lax.broadcasted_iota(i32, (1, _N), 1)


    def p1_cond(carry):
        i, H, pivot = carry
        return (i < _N) & (pivot > 0)

    def p1_step(carry):
        i, H, pivot = carry
        col = _N - 1 - i
        c16 = jnp.sum(jnp.where(lane_n == col, H, 0),
                      axis=1, keepdims=True)
        bits = (c16 >> bit_iota) & 1
        cand = jnp.where((bits != 0) & (rowid < pivot), rowid, -1)
        max_row = jnp.max(cand)
        found = max_row >= 0
        new_pivot = pivot - 1
        r1 = jnp.where(found, max_row, 0)
        r2 = jnp.where(found, new_pivot, 0)
        w1, b1 = r1 >> 5, r1 & 31
        w2, b2 = r2 >> 5, r2 & 31
        comb = jnp.sum(jnp.where(iota_w == w1, (H >> b1) & 1, 0)
                       + jnp.where(iota_w == w2, ((H >> b2) & 1) << 1, 0),
                       axis=0, keepdims=True)
        r1bit = comb & 1
        d = jnp.where(found, (comb ^ (comb >> 1)) & 1, 0)
        H = (H ^ jnp.where(iota_w == w1, d << b1, 0)
               ^ jnp.where(iota_w == w2, d << b2, 0))
        rowmask = 0 - r1bit
        clr1 = jnp.where(iota_w == w1, i32(1) << b1, 0)
        clr2 = jnp.where(iota_w == w2, i32(1) << b2, 0)
        pc16 = jnp.where(found, c16 & ~clr1 & ~clr2, 0)
        H = H ^ (pc16 & rowmask)
        return i + 1, H, jnp.where(found, new_pivot, pivot)

    _, Hf, _ = lax.while_loop(p1_cond, p1_step, (i32(0), H0, i32(_NR)))

    wnz = Hf != 0
    single = (Hf & (Hf - 1)) == 0
    all_single = jnp.all(single, axis=0, keepdims=True)
    nzcnt = jnp.sum(wnz.astype(i32), axis=0, keepdims=True)
    posw = (((Hf & i32(-1431655766)) != 0).astype(i32)
            + (((Hf & i32(-858993460)) != 0).astype(i32) << 1)
            + (((Hf & i32(-252645136)) != 0).astype(i32) << 2)
            + (((Hf & i32(-16711936)) != 0).astype(i32) << 3)
            + (((Hf & i32(-65536)) != 0).astype(i32) << 4))
    rowpos = jnp.sum(jnp.where(wnz, iota_w * 32 + posw, 0),
                     axis=0, keepdims=True)
    colid0 = jnp.where(all_single & (nzcnt == 1), rowpos, -1)

    def p2_body(i, carry):
        colid, csv = carry
        m = colid == i
        cond = jnp.any(m)
        jstar = jnp.min(jnp.where(m, lane_n, i32(1) << 20))
        jstar = jnp.where(cond, jstar, 0)
        cei = _N - _NR + i
        mA = lane_n == jstar
        mB = lane_n == cei
        cidB = jnp.sum(jnp.where(mB, colid, 0))
        cid_sw = jnp.where(mA, cidB, jnp.where(mB, i, colid))
        colid = jnp.where(cond, cid_sw, colid)
        sA = jnp.sum(jnp.where(mA, csv, 0))
        sB = jnp.sum(jnp.where(mB, csv, 0))
        cs_sw = jnp.where(mA, sB, jnp.where(mB, sA, csv))
        return colid, jnp.where(cond, cs_sw, csv)

    _, csv = lax.fori_loop(
        0, _NR, p2_body, (colid0, lax.broadcasted_iota(i32, (1, _N), 1)))

    for w in range(_NW):
        g_ref[pl.ds(32 * w, 32), :] = (
            (Hf[w:w + 1, :] >> sh32) & 1).astype(f32)
    perm = (lax.broadcasted_iota(i32, (_N, _NR), 0)
            == csv[:, 0:_NR]).astype(f32)
    u_ref[:, :] = lax.dot_general(g_ref[:, :], perm,
                                  (((1,), (0,)), ((), ())),
                                  preferred_element_type=f32)
    csf = csv.astype(f32)
    eye = (lax.broadcasted_iota(i32, (_NR, _NR), 0)
           == lax.broadcasted_iota(i32, (_NR, _NR), 1)).astype(f32)
    cs_top = lax.dot_general(eye, csf[:, 0:_NR],
                             (((1,), (1,)), ((), ())),
                             preferred_element_type=f32)
    cs_bot = lax.dot_general(eye, csf[:, _NR:_N],
                             (((1,), (1,)), ((), ())),
                             preferred_element_type=f32)
    lane_ni = lax.broadcasted_iota(i32, (_NR, _N), 1)
    m_ref[:, :] = (cs_bot.astype(i32) == lane_ni).astype(f32)
    g_ref[:, :] = ((cs_top.astype(i32) == lane_ni).astype(f32)
                   + lax.dot_general(u_ref[:, :], m_ref[:, :],
                                     (((0,), (0,)), ((), ())),
                                     preferred_element_type=f32))
    src_ref[0:1, :] = csv[0:1, 0:_NR]


def kernel(H_input):
    g, src = pl.pallas_call(
        _gj_body,
        out_shape=[
            jax.ShapeDtypeStruct((_NR, _N), jnp.float32),
            jax.ShapeDtypeStruct((1, _NR), jnp.int32),
        ],
        scratch_shapes=[
            pltpu.VMEM((_NR, _NR), jnp.float32),
            pltpu.VMEM((_NR, _N), jnp.float32),
        ],
    )(H_input)
    return g, src.reshape(_NR)


# speedup vs baseline: 29.0643x; 1.0011x over previous
import jax
import jax.numpy as jnp
from jax import lax
from jax.experimental import pallas as pl
from jax.experimental.pallas import tpu as pltpu

_N = 1024
_NR = 512
_NW = 16


def _gj_body(hin_ref, g_ref, src_ref, u_ref, m_ref):
    i32 = jnp.int32
    f32 = jnp.float32

    sh32 = lax.broadcasted_iota(i32, (32, 1), 0)
    H0 = jnp.concatenate(
        [jnp.sum(hin_ref[pl.ds(32 * w, 32), :].astype(i32) << sh32,
                 axis=0, keepdims=True) for w in range(_NW)], axis=0)

    iota_w = lax.broadcasted_iota(i32, (_NW, 1), 0)
    bit_iota = lax.broadcasted_iota(i32, (_NW, 32), 1)
    rowid = lax.broadcasted_iota(i32, (_NW, 32), 0) * 32 + bit_iota
    lane_n = lax.broadcasted_iota(i32, (1, _N), 1)


    def p1_cond(carry):
        i, H, pivot = carry
        return (i < _N) & (pivot > 0)

    def p1_step(carry):
        i, H, pivot = carry
        col = _N - 1 - i
        c16 = jnp.sum(jnp.where(lane_n == col, H, 0),
                      axis=1, keepdims=True)
        bits = (c16 >> bit_iota) & 1
        cand = jnp.where((bits != 0) & (rowid < pivot), rowid, -1)
        max_row = jnp.max(cand)
        found = max_row >= 0
        new_pivot = pivot - 1
        r1 = jnp.where(found, max_row, 0)
        r2 = jnp.where(found, new_pivot, 0)
        w1, b1 = r1 >> 5, r1 & 31
        w2, b2 = r2 >> 5, r2 & 31
        comb = jnp.sum(jnp.where(iota_w == w1, (H >> b1) & 1, 0)
                       + jnp.where(iota_w == w2, ((H >> b2) & 1) << 1, 0),
                       axis=0, keepdims=True)
        r1bit = comb & 1
        d = jnp.where(found, (comb ^ (comb >> 1)) & 1, 0)
        H = (H ^ jnp.where(iota_w == w1, d << b1, 0)
               ^ jnp.where(iota_w == w2, d << b2, 0))
        rowmask = 0 - r1bit
        clr1 = jnp.where(iota_w == w1, i32(1) << b1, 0)
        clr2 = jnp.where(iota_w == w2, i32(1) << b2, 0)
        pc16 = jnp.where(found, c16 & ~clr1 & ~clr2, 0)
        H = H ^ (pc16 & rowmask)
        return i + 1, H, jnp.where(found, new_pivot, pivot)

    _, Hf, _ = lax.while_loop(p1_cond, p1_step, (i32(0), H0, i32(_NR)))

    wnz = Hf != 0
    single = (Hf & (Hf - 1)) == 0
    all_single = jnp.all(single, axis=0, keepdims=True)
    nzcnt = jnp.sum(wnz.astype(i32), axis=0, keepdims=True)
    posw = (((Hf & i32(-1431655766)) != 0).astype(i32)
            + (((Hf & i32(-858993460)) != 0).astype(i32) << 1)
            + (((Hf & i32(-252645136)) != 0).astype(i32) << 2)
            + (((Hf & i32(-16711936)) != 0).astype(i32) << 3)
            + (((Hf & i32(-65536)) != 0).astype(i32) << 4))
    rowpos = jnp.sum(jnp.where(wnz, iota_w * 32 + posw, 0),
                     axis=0, keepdims=True)
    colid0 = jnp.where(all_single & (nzcnt == 1), rowpos, -1)

    big = i32(1) << 20
    i_col = lax.broadcasted_iota(i32, (_NR, 1), 0)
    fm = jnp.min(jnp.where(colid0 == i_col, lane_n, big),
                 axis=1, keepdims=True)
    ok = (fm == _N - _NR + i_col) | (fm >= big)
    i0 = jnp.min(jnp.where(ok, _NR, i_col))

    def p2_cond(carry):
        return carry[0] < _NR

    def p2_body(carry):
        i, colid, csv = carry
        m = colid == i
        cond = jnp.any(m)
        jstar = jnp.min(jnp.where(m, lane_n, big))
        jstar = jnp.where(cond, jstar, 0)
        cei = _N - _NR + i
        mA = lane_n == jstar
        mB = lane_n == cei
        cidB = jnp.sum(jnp.where(mB, colid, 0))
        cid_sw = jnp.where(mA, cidB, jnp.where(mB, i, colid))
        colid = jnp.where(cond, cid_sw, colid)
        sA = jnp.sum(jnp.where(mA, csv, 0))
        sB = jnp.sum(jnp.where(mB, csv, 0))
        cs_sw = jnp.where(mA, sB, jnp.where(mB, sA, csv))
        return i + 1, colid, jnp.where(cond, cs_sw, csv)

    _, _, csv = lax.while_loop(
        p2_cond, p2_body,
        (i0, colid0, lax.broadcasted_iota(i32, (1, _N), 1)))

    for w in range(_NW):
        g_ref[pl.ds(32 * w, 32), :] = (
            (Hf[w:w + 1, :] >> sh32) & 1).astype(f32)
    perm = (lax.broadcasted_iota(i32, (_N, _NR), 0)
            == csv[:, 0:_NR]).astype(f32)
    u_ref[:, :] = lax.dot_general(g_ref[:, :], perm,
                                  (((1,), (0,)), ((), ())),
                                  preferred_element_type=f32)
    csf = csv.astype(f32)
    eye = (lax.broadcasted_iota(i32, (_NR, _NR), 0)
           == lax.broadcasted_iota(i32, (_NR, _NR), 1)).astype(f32)
    cs_top = lax.dot_general(eye, csf[:, 0:_NR],
                             (((1,), (1,)), ((), ())),
                             preferred_element_type=f32)
    cs_bot = lax.dot_general(eye, csf[:, _NR:_N],
                             (((1,), (1,)), ((), ())),
                             preferred_element_type=f32)
    lane_ni = lax.broadcasted_iota(i32, (_NR, _N), 1)
    m_ref[:, :] = (cs_bot.astype(i32) == lane_ni).astype(f32)
    g_ref[:, :] = ((cs_top.astype(i32) == lane_ni).astype(f32)
                   + lax.dot_general(u_ref[:, :], m_ref[:, :],
                                     (((0,), (0,)), ((), ())),
                                     preferred_element_type=f32))
    src_ref[0:1, :] = csv[0:1, 0:_NR]


def kernel(H_input):
    g, src = pl.pallas_call(
        _gj_body,
        out_shape=[
            jax.ShapeDtypeStruct((_NR, _N), jnp.float32),
            jax.ShapeDtypeStruct((1, _NR), jnp.int32),
        ],
        scratch_shapes=[
            pltpu.VMEM((_NR, _NR), jnp.float32),
            pltpu.VMEM((_NR, _N), jnp.float32),
        ],
    )(H_input)
    return g, src.reshape(_NR)
